# Initial kernel scaffold; baseline (speedup 1.0000x reference)
import jax
import jax.numpy as jnp
from jax.experimental import pallas as pl


def kernel(inputs, table):
    raise NotImplementedError("write your pallas kernel here")


# speedup vs baseline: 4.4788x; 4.4788x over previous
import functools

import jax
import jax.numpy as jnp
from jax import lax
from jax.experimental import pallas as pl
from jax.experimental.pallas import tpu as pltpu
from jax.experimental.pallas import tpu_sc as plsc

_NUM_CORES = 2
_NUM_SUBCORES = 16
_NUM_WORKERS = _NUM_CORES * _NUM_SUBCORES


@functools.lru_cache(maxsize=None)
def _build_gather(batch: int, dim: int):
    assert batch % _NUM_WORKERS == 0
    b_per_w = batch // _NUM_WORKERS
    chunk = 640
    while b_per_w % chunk != 0:
        chunk //= 2
    n_chunks = b_per_w // chunk

    mesh = plsc.VectorSubcoreMesh(core_axis_name="c", subcore_axis_name="s")

    @functools.partial(
        pl.kernel,
        mesh=mesh,
        out_type=jax.ShapeDtypeStruct((batch, dim), jnp.float32),
        scratch_types=[
            pltpu.VMEM((chunk,), jnp.int32),
            pltpu.VMEM((chunk, dim), jnp.float32),
            pltpu.SemaphoreType.DMA,
        ],
        compiler_params=pltpu.CompilerParams(use_tc_tiling_on_sc=False),
    )
    def gather_kernel(idx_hbm, table_hbm, out_hbm, idx_v, rows_v, sem):
        wid = lax.axis_index("s") * _NUM_CORES + lax.axis_index("c")
        base = wid * b_per_w
        for c in range(n_chunks):
            off = base + c * chunk
            pltpu.sync_copy(idx_hbm.at[pl.ds(off, chunk)], idx_v)
            pltpu.async_copy(table_hbm.at[idx_v], rows_v, sem).wait()
            pltpu.sync_copy(rows_v, out_hbm.at[pl.ds(off, chunk)])

    return gather_kernel


def kernel(inputs, table):
    batch, hist = inputs.shape
    vocab, dim = table.shape
    idx = inputs.reshape(-1)
    out = _build_gather(batch * hist, dim)(idx, table)
    return (out.reshape(batch, hist, dim), inputs)


# speedup vs baseline: 4.6617x; 1.0408x over previous
import functools

import jax
import jax.numpy as jnp
from jax import lax
from jax.experimental import pallas as pl
from jax.experimental.pallas import tpu as pltpu
from jax.experimental.pallas import tpu_sc as plsc

_NUM_CORES = 2
_NUM_SUBCORES = 16
_NUM_WORKERS = _NUM_CORES * _NUM_SUBCORES


@functools.lru_cache(maxsize=None)
def _build_gather(batch: int, dim: int):
    assert batch % _NUM_WORKERS == 0
    b_per_w = batch // _NUM_WORKERS
    chunk = 800
    while b_per_w % chunk != 0:
        chunk //= 2
    n_chunks = b_per_w // chunk
    nbuf = 2

    mesh = plsc.VectorSubcoreMesh(core_axis_name="c", subcore_axis_name="s")

    @functools.partial(
        pl.kernel,
        mesh=mesh,
        out_type=jax.ShapeDtypeStruct((batch, dim), jnp.float32),
        scratch_types=[
            pltpu.VMEM((b_per_w,), jnp.int32),
            pltpu.VMEM((nbuf, chunk, dim), jnp.float32),
            pltpu.SemaphoreType.DMA((nbuf,)),
            pltpu.SemaphoreType.DMA((nbuf,)),
        ],
        compiler_params=pltpu.CompilerParams(use_tc_tiling_on_sc=False),
    )
    def gather_kernel(idx_hbm, table_hbm, out_hbm, idx_v, rows_v, gsem, ssem):
        wid = lax.axis_index("s") * _NUM_CORES + lax.axis_index("c")
        base = wid * b_per_w
        pltpu.sync_copy(idx_hbm.at[pl.ds(base, b_per_w)], idx_v)
        gathers = [None] * n_chunks
        stores = [None] * n_chunks
        for c in range(n_chunks):
            b = c % nbuf
            if c >= nbuf:
                stores[c - nbuf].wait()
            gathers[c] = pltpu.async_copy(
                table_hbm.at[idx_v.at[pl.ds(c * chunk, chunk)]],
                rows_v.at[b],
                gsem.at[b],
            )
            if c >= 1:
                pb = (c - 1) % nbuf
                gathers[c - 1].wait()
                stores[c - 1] = pltpu.async_copy(
                    rows_v.at[pb],
                    out_hbm.at[pl.ds(base + (c - 1) * chunk, chunk)],
                    ssem.at[pb],
                )
        gathers[n_chunks - 1].wait()
        stores[n_chunks - 1] = pltpu.async_copy(
            rows_v.at[(n_chunks - 1) % nbuf],
            out_hbm.at[pl.ds(base + (n_chunks - 1) * chunk, chunk)],
            ssem.at[(n_chunks - 1) % nbuf],
        )
        stores[n_chunks - 2].wait()
        stores[n_chunks - 1].wait()

    return gather_kernel


def kernel(inputs, table):
    batch, hist = inputs.shape
    vocab, dim = table.shape
    idx = inputs.reshape(-1)
    out = _build_gather(batch * hist, dim)(idx, table)
    return (out.reshape(batch, hist, dim), inputs)
